# baseline (speedup 1.0000x reference)
import jax
import jax.numpy as jnp
from jax import lax
from jax.experimental import pallas as pl
from jax.experimental.pallas import tpu as pltpu
from jax.experimental.pallas import tpu_sc as plsc

N = 50000
D = 256
B = 128
NC = 2
NS = 16
NW = NC * NS
T = 96
G = 16
SEG = 2 * B
ACC_ROWS = SEG + 16
NTC = 24000
NSC8 = (N - NTC) // 8
CHUNK = 8 * ((NSC8 + NW - 1) // NW + 1)
_NT_RAW = (CHUNK + T - 1) // T
N_TILES = _NT_RAW + (_NT_RAW % 2)
N_PAIRS = N_TILES // 2
RB = 2000
NB = NTC // RB
W = 64


def _worker_base(w):
    return NTC + 8 * ((w * NSC8) // NW)


def _sc_body(out_hbm, zv_hbm, bat_hbm, parts_hbm,
             rowa, rowb, zvbuf, batbuf, acc, sema, semb):
    c = lax.axis_index("c")
    s = lax.axis_index("s")
    w = c * NS + s

    base = _worker_base(w)
    wend = _worker_base(w + 1)
    cb = jnp.minimum(base, N - CHUNK)
    lane = lax.iota(jnp.int32, 16)
    zero16 = jnp.zeros((16,), jnp.float32)

    def tile_base(j):
        return jnp.minimum(base + j * T, wend - T)

    pltpu.async_copy(out_hbm.at[pl.ds(tile_base(0), T)], rowa, sema)
    pltpu.async_copy(out_hbm.at[pl.ds(tile_base(1), T)], rowb, semb)
    pltpu.sync_copy(zv_hbm.at[pl.ds(cb, CHUNK)], zvbuf)
    pltpu.sync_copy(bat_hbm.at[pl.ds(cb, CHUNK)], batbuf)

    def zero_body(i, carry):
        for k in range(16):
            acc[pl.ds(i * 256 + k * 16, 16)] = zero16
        return carry

    lax.fori_loop(0, ACC_ROWS, zero_body, 0)

    def flush(carry):
        cur_b, srs, sas = carry
        off_r = 2 * jnp.maximum(cur_b, 0) * D
        for jj in range(D // 16):
            o1 = off_r + jj * 16
            acc[pl.ds(o1, 16)] = acc[pl.ds(o1, 16)] + srs[jj]
            o2 = off_r + D + jj * 16
            acc[pl.ds(o2, 16)] = acc[pl.ds(o2, 16)] + (sas[jj] - srs[jj])
        zeros = tuple(zero16 for _ in range(D // 16))
        return zeros, zeros

    def process(buf, j, carry):
        tb = tile_base(j)
        delta = base + j * T - tb
        rel = tb - cb

        def grp_body(g, carry):
            z = zvbuf[pl.ds(rel + g * G, 16)]
            bt = batbuf[pl.ds(rel + g * G, 16)]
            fast = (bt[0] == bt[15]) & ((g * G) >= delta)

            def fast_fn(carry):
                cur_b, srs, sas = carry
                b = bt[0]

                def keep(carry):
                    return carry[1], carry[2]

                srs, sas = lax.cond(b != cur_b, flush, keep,
                                    (cur_b, srs, sas))
                srl, sal = list(srs), list(sas)
                for r in range(G):
                    mreal = jnp.full((16,), z[r], jnp.int32) != 100
                    for jj in range(D // 16):
                        v = buf[g * G + r, pl.ds(jj * 16, 16)]
                        sal[jj] = sal[jj] + v
                        srl[jj] = srl[jj] + jnp.where(mreal, v, 0.0)
                return b, tuple(srl), tuple(sal)

            def slow_fn(carry):
                segv = bt * 2 + jnp.where(z == 100, 1, 0)
                valid = (lane + g * G) >= delta
                segv = jnp.where(valid, segv * D, SEG * D)
                for r in range(G):
                    seg16 = jnp.full((16,), segv[r], jnp.int32)
                    for jj in range(D // 16):
                        val = buf[g * G + r, pl.ds(jj * 16, 16)]
                        plsc.addupdate_scatter(
                            acc, [seg16 + (jj * 16) + lane], val)
                return carry

            return lax.cond(fast, fast_fn, slow_fn, carry)

        return lax.fori_loop(0, T // G, grp_body, carry)

    def pair_body(p, carry):
        j0 = 2 * p
        pltpu.make_async_copy(out_hbm.at[pl.ds(0, T)], rowa, sema).wait()
        carry = process(rowa, j0, carry)

        @pl.when(j0 + 2 < N_TILES)
        def _next_a():
            pltpu.async_copy(out_hbm.at[pl.ds(tile_base(j0 + 2), T)],
                             rowa, sema)

        pltpu.make_async_copy(out_hbm.at[pl.ds(0, T)], rowb, semb).wait()
        carry = process(rowb, j0 + 1, carry)

        @pl.when(j0 + 3 < N_TILES)
        def _next_b():
            pltpu.async_copy(out_hbm.at[pl.ds(tile_base(j0 + 3), T)],
                             rowb, semb)

        return carry

    zeros0 = tuple(zero16 for _ in range(D // 16))
    carry = (jnp.int32(-1), zeros0, zeros0)
    carry = lax.fori_loop(0, N_PAIRS, pair_body, carry)
    flush(carry)

    pltpu.sync_copy(acc.at[pl.ds(0, SEG * D)], parts_hbm.at[w])


_sc_pool = pl.kernel(
    _sc_body,
    out_type=jax.ShapeDtypeStruct((NW, SEG * D), jnp.float32),
    mesh=plsc.VectorSubcoreMesh(core_axis_name="c", subcore_axis_name="s"),
    compiler_params=pltpu.CompilerParams(needs_layout_passes=False),
    scratch_types=[
        pltpu.VMEM((T, D), jnp.float32),
        pltpu.VMEM((T, D), jnp.float32),
        pltpu.VMEM((CHUNK,), jnp.int32),
        pltpu.VMEM((CHUNK,), jnp.int32),
        pltpu.VMEM((ACC_ROWS * D,), jnp.float32),
        pltpu.SemaphoreType.DMA,
        pltpu.SemaphoreType.DMA,
    ],
)


def _tc_body(zv_ref, bat_ref, rows_ref, parts_ref, o_ref):
    i = pl.program_id(0)
    zvb = zv_ref[0, 0, :]
    btb = bat_ref[0, 0, :]
    sv = btb * 2 + jnp.where(zvb == 100, 1, 0)
    w0 = jnp.minimum((btb[0] * 2) // 8 * 8, SEG - W)
    w0 = pl.multiple_of(w0, 8)
    svw = sv - w0
    rows_bf = rows_ref[...].astype(jnp.bfloat16)

    @pl.when(i == 0)
    def _init():
        o_ref[...] = jnp.zeros((SEG, D), jnp.float32)

    oh = (lax.broadcasted_iota(jnp.int32, (W, RB), 0)
          == svw[None, :]).astype(jnp.bfloat16)
    o_ref[pl.ds(w0, W), :] += jnp.dot(oh, rows_bf,
                                      preferred_element_type=jnp.float32)

    @pl.when(jnp.any(svw >= W))
    def _fallback():
        ohf = ((lax.broadcasted_iota(jnp.int32, (SEG, RB), 0) == sv[None, :])
               & (svw >= W)[None, :]).astype(jnp.bfloat16)
        o_ref[...] += jnp.dot(ohf, rows_bf,
                              preferred_element_type=jnp.float32)

    @pl.when(i == NB - 1)
    def _add_sc_partials():
        acc = parts_ref[0]
        for k in range(1, NW):
            acc = acc + parts_ref[k]
        o_ref[...] += acc


_tc_pool = pl.pallas_call(
    _tc_body,
    grid=(NB,),
    in_specs=[
        pl.BlockSpec((1, 1, RB), lambda i: (i, 0, 0)),
        pl.BlockSpec((1, 1, RB), lambda i: (i, 0, 0)),
        pl.BlockSpec((RB, D), lambda i: (i, 0)),
        pl.BlockSpec((NW, SEG, D), lambda i: (0, 0, 0)),
    ],
    out_specs=pl.BlockSpec((SEG, D), lambda i: (0, 0)),
    out_shape=jax.ShapeDtypeStruct((SEG, D), jnp.float32),
)


def kernel(out, zv, x_rv_batch):
    zv32 = zv.astype(jnp.int32)
    bat32 = x_rv_batch.astype(jnp.int32)
    parts = _sc_pool(out, zv32, bat32)
    final = _tc_pool(zv32[:NTC].reshape(NB, 1, RB),
                     bat32[:NTC].reshape(NB, 1, RB),
                     out[:NTC],
                     parts.reshape(NW, SEG, D))
    return final.reshape(B, 2 * D)


# speedup vs baseline: 1.0342x; 1.0342x over previous
import jax
import jax.numpy as jnp
from jax import lax
from jax.experimental import pallas as pl
from jax.experimental.pallas import tpu as pltpu
from jax.experimental.pallas import tpu_sc as plsc

N = 50000
D = 256
B = 128
NC = 2
NS = 16
NW = NC * NS
T = 96
G = 16
SEG = 2 * B
ACC_ROWS = SEG + 16
NTC = 24000
NSC8 = (N - NTC) // 8
CHUNK = 8 * ((NSC8 + NW - 1) // NW + 1)
_NT_RAW = (CHUNK + T - 1) // T
N_TILES = _NT_RAW + (_NT_RAW % 2)
N_PAIRS = N_TILES // 2
RB = 3000
NB = NTC // RB
PPS = NW // NB
W = 64


def _worker_base(w):
    return NTC + 8 * ((w * NSC8) // NW)


def _sc_body(out_hbm, zv_hbm, bat_hbm, parts_hbm,
             rowa, rowb, zvbuf, batbuf, acc, sema, semb):
    c = lax.axis_index("c")
    s = lax.axis_index("s")
    w = c * NS + s

    base = _worker_base(w)
    wend = _worker_base(w + 1)
    cb = jnp.minimum(base, N - CHUNK)
    lane = lax.iota(jnp.int32, 16)
    zero16 = jnp.zeros((16,), jnp.float32)

    def tile_base(j):
        return jnp.minimum(base + j * T, wend - T)

    pltpu.async_copy(out_hbm.at[pl.ds(tile_base(0), T)], rowa, sema)
    pltpu.async_copy(out_hbm.at[pl.ds(tile_base(1), T)], rowb, semb)
    pltpu.sync_copy(zv_hbm.at[pl.ds(cb, CHUNK)], zvbuf)
    pltpu.sync_copy(bat_hbm.at[pl.ds(cb, CHUNK)], batbuf)

    def zero_body(i, carry):
        for k in range(16):
            acc[pl.ds(i * 256 + k * 16, 16)] = zero16
        return carry

    lax.fori_loop(0, ACC_ROWS, zero_body, 0)

    def flush(carry):
        cur_b, srs, sas = carry
        off_r = 2 * jnp.maximum(cur_b, 0) * D
        for jj in range(D // 16):
            o1 = off_r + jj * 16
            acc[pl.ds(o1, 16)] = acc[pl.ds(o1, 16)] + srs[jj]
            o2 = off_r + D + jj * 16
            acc[pl.ds(o2, 16)] = acc[pl.ds(o2, 16)] + (sas[jj] - srs[jj])
        zeros = tuple(zero16 for _ in range(D // 16))
        return zeros, zeros

    def process(buf, j, carry):
        tb = tile_base(j)
        delta = base + j * T - tb
        rel = tb - cb

        def grp_body(g, carry):
            z = zvbuf[pl.ds(rel + g * G, 16)]
            bt = batbuf[pl.ds(rel + g * G, 16)]
            fast = (bt[0] == bt[15]) & ((g * G) >= delta)

            def fast_fn(carry):
                cur_b, srs, sas = carry
                b = bt[0]

                def keep(carry):
                    return carry[1], carry[2]

                srs, sas = lax.cond(b != cur_b, flush, keep,
                                    (cur_b, srs, sas))
                srl, sal = list(srs), list(sas)
                for r in range(G):
                    mreal = jnp.full((16,), z[r], jnp.int32) != 100
                    for jj in range(D // 16):
                        v = buf[g * G + r, pl.ds(jj * 16, 16)]
                        sal[jj] = sal[jj] + v
                        srl[jj] = srl[jj] + jnp.where(mreal, v, 0.0)
                return b, tuple(srl), tuple(sal)

            def slow_fn(carry):
                segv = bt * 2 + jnp.where(z == 100, 1, 0)
                valid = (lane + g * G) >= delta
                segv = jnp.where(valid, segv * D, SEG * D)
                for r in range(G):
                    seg16 = jnp.full((16,), segv[r], jnp.int32)
                    for jj in range(D // 16):
                        val = buf[g * G + r, pl.ds(jj * 16, 16)]
                        plsc.addupdate_scatter(
                            acc, [seg16 + (jj * 16) + lane], val)
                return carry

            return lax.cond(fast, fast_fn, slow_fn, carry)

        return lax.fori_loop(0, T // G, grp_body, carry)

    def pair_body(p, carry):
        j0 = 2 * p
        pltpu.make_async_copy(out_hbm.at[pl.ds(0, T)], rowa, sema).wait()
        carry = process(rowa, j0, carry)

        @pl.when(j0 + 2 < N_TILES)
        def _next_a():
            pltpu.async_copy(out_hbm.at[pl.ds(tile_base(j0 + 2), T)],
                             rowa, sema)

        pltpu.make_async_copy(out_hbm.at[pl.ds(0, T)], rowb, semb).wait()
        carry = process(rowb, j0 + 1, carry)

        @pl.when(j0 + 3 < N_TILES)
        def _next_b():
            pltpu.async_copy(out_hbm.at[pl.ds(tile_base(j0 + 3), T)],
                             rowb, semb)

        return carry

    zeros0 = tuple(zero16 for _ in range(D // 16))
    carry = (jnp.int32(-1), zeros0, zeros0)
    carry = lax.fori_loop(0, N_PAIRS, pair_body, carry)
    flush(carry)

    pltpu.sync_copy(acc.at[pl.ds(0, SEG * D)], parts_hbm.at[w])


_sc_pool = pl.kernel(
    _sc_body,
    out_type=jax.ShapeDtypeStruct((NW, SEG * D), jnp.float32),
    mesh=plsc.VectorSubcoreMesh(core_axis_name="c", subcore_axis_name="s"),
    compiler_params=pltpu.CompilerParams(needs_layout_passes=False),
    scratch_types=[
        pltpu.VMEM((T, D), jnp.float32),
        pltpu.VMEM((T, D), jnp.float32),
        pltpu.VMEM((CHUNK,), jnp.int32),
        pltpu.VMEM((CHUNK,), jnp.int32),
        pltpu.VMEM((ACC_ROWS * D,), jnp.float32),
        pltpu.SemaphoreType.DMA,
        pltpu.SemaphoreType.DMA,
    ],
)


def _tc_body(zv_ref, bat_ref, rows_ref, parts_ref, o_ref):
    i = pl.program_id(0)
    zvb = zv_ref[0, 0, :]
    btb = bat_ref[0, 0, :]
    sv = btb * 2 + jnp.where(zvb == 100, 1, 0)
    w0 = jnp.minimum((btb[0] * 2) // 8 * 8, SEG - W)
    w0 = pl.multiple_of(w0, 8)
    svw = sv - w0
    rows_bf = rows_ref[...].astype(jnp.bfloat16)

    @pl.when(i == 0)
    def _init():
        o_ref[...] = jnp.zeros((SEG, D), jnp.float32)

    oh = (lax.broadcasted_iota(jnp.int32, (W, RB), 0)
          == svw[None, :]).astype(jnp.bfloat16)
    o_ref[pl.ds(w0, W), :] += jnp.dot(oh, rows_bf,
                                      preferred_element_type=jnp.float32)
    psum = parts_ref[0]
    for k in range(1, PPS):
        psum = psum + parts_ref[k]
    o_ref[...] += psum

    @pl.when(jnp.any(svw >= W))
    def _fallback():
        ohf = ((lax.broadcasted_iota(jnp.int32, (SEG, RB), 0) == sv[None, :])
               & (svw >= W)[None, :]).astype(jnp.bfloat16)
        o_ref[...] += jnp.dot(ohf, rows_bf,
                              preferred_element_type=jnp.float32)


_tc_pool = pl.pallas_call(
    _tc_body,
    grid=(NB,),
    in_specs=[
        pl.BlockSpec((1, 1, RB), lambda i: (i, 0, 0)),
        pl.BlockSpec((1, 1, RB), lambda i: (i, 0, 0)),
        pl.BlockSpec((RB, D), lambda i: (i, 0)),
        pl.BlockSpec((PPS, SEG, D), lambda i: (i, 0, 0)),
    ],
    out_specs=pl.BlockSpec((SEG, D), lambda i: (0, 0)),
    out_shape=jax.ShapeDtypeStruct((SEG, D), jnp.float32),
)


def kernel(out, zv, x_rv_batch):
    zv32 = zv.astype(jnp.int32)
    bat32 = x_rv_batch.astype(jnp.int32)
    parts = _sc_pool(out, zv32, bat32)
    final = _tc_pool(zv32[:NTC].reshape(NB, 1, RB),
                     bat32[:NTC].reshape(NB, 1, RB),
                     out[:NTC],
                     parts.reshape(NW, SEG, D))
    return final.reshape(B, 2 * D)


# speedup vs baseline: 1.4322x; 1.3848x over previous
import jax
import jax.numpy as jnp
from jax import lax
from jax.experimental import pallas as pl
from jax.experimental.pallas import tpu as pltpu
from jax.experimental.pallas import tpu_sc as plsc

N = 50000
D = 256
B = 128
NC = 2
NS = 16
NW = NC * NS
T = 96
G = 16
SEG = 2 * B
ACC_ROWS = SEG + 16
NTC = 28000
NSC8 = (N - NTC) // 8
CHUNK = 8 * ((NSC8 + NW - 1) // NW + 1)
_NT_RAW = (CHUNK + T - 1) // T
N_TILES = _NT_RAW + (_NT_RAW % 2)
N_PAIRS = N_TILES // 2
RB = 2000
NB = NTC // RB
W = 64


def _worker_base(w):
    return NTC + 8 * ((w * NSC8) // NW)


def _sc_body(out_hbm, zv_hbm, bat_hbm, parts_hbm,
             rowa, rowb, zvbuf, batbuf, acc, sema, semb):
    c = lax.axis_index("c")
    s = lax.axis_index("s")
    w = c * NS + s

    base = _worker_base(w)
    wend = _worker_base(w + 1)
    cb = jnp.minimum(base, N - CHUNK)
    lane = lax.iota(jnp.int32, 16)
    zero16 = jnp.zeros((16,), jnp.float32)

    def tile_base(j):
        return jnp.minimum(base + j * T, wend - T)

    pltpu.async_copy(out_hbm.at[pl.ds(tile_base(0), T)], rowa, sema)
    pltpu.async_copy(out_hbm.at[pl.ds(tile_base(1), T)], rowb, semb)
    pltpu.sync_copy(zv_hbm.at[pl.ds(cb, CHUNK)], zvbuf)
    pltpu.sync_copy(bat_hbm.at[pl.ds(cb, CHUNK)], batbuf)

    def zero_body(i, carry):
        for k in range(16):
            acc[pl.ds(i * 256 + k * 16, 16)] = zero16
        return carry

    lax.fori_loop(0, ACC_ROWS, zero_body, 0)

    def flush(carry):
        cur_b, srs, sas = carry
        off_r = 2 * jnp.maximum(cur_b, 0) * D
        for jj in range(D // 16):
            o1 = off_r + jj * 16
            acc[pl.ds(o1, 16)] = acc[pl.ds(o1, 16)] + srs[jj]
            o2 = off_r + D + jj * 16
            acc[pl.ds(o2, 16)] = acc[pl.ds(o2, 16)] + (sas[jj] - srs[jj])
        zeros = tuple(zero16 for _ in range(D // 16))
        return zeros, zeros

    def process(buf, j, carry):
        tb = tile_base(j)
        delta = base + j * T - tb
        rel = tb - cb

        def grp_body(g, carry):
            z = zvbuf[pl.ds(rel + g * G, 16)]
            bt = batbuf[pl.ds(rel + g * G, 16)]
            fast = (bt[0] == bt[15]) & ((g * G) >= delta)

            def fast_fn(carry):
                cur_b, srs, sas = carry
                b = bt[0]

                def keep(carry):
                    return carry[1], carry[2]

                srs, sas = lax.cond(b != cur_b, flush, keep,
                                    (cur_b, srs, sas))
                srl, sal = list(srs), list(sas)
                for r in range(G):
                    mreal = jnp.full((16,), z[r], jnp.int32) != 100
                    for jj in range(D // 16):
                        v = buf[g * G + r, pl.ds(jj * 16, 16)]
                        sal[jj] = sal[jj] + v
                        srl[jj] = srl[jj] + jnp.where(mreal, v, 0.0)
                return b, tuple(srl), tuple(sal)

            def slow_fn(carry):
                segv = bt * 2 + jnp.where(z == 100, 1, 0)
                valid = (lane + g * G) >= delta
                segv = jnp.where(valid, segv * D, SEG * D)
                for r in range(G):
                    seg16 = jnp.full((16,), segv[r], jnp.int32)
                    for jj in range(D // 16):
                        val = buf[g * G + r, pl.ds(jj * 16, 16)]
                        plsc.addupdate_scatter(
                            acc, [seg16 + (jj * 16) + lane], val)
                return carry

            return lax.cond(fast, fast_fn, slow_fn, carry)

        return lax.fori_loop(0, T // G, grp_body, carry)

    def pair_body(p, carry):
        j0 = 2 * p
        pltpu.make_async_copy(out_hbm.at[pl.ds(0, T)], rowa, sema).wait()
        carry = process(rowa, j0, carry)

        @pl.when(j0 + 2 < N_TILES)
        def _next_a():
            pltpu.async_copy(out_hbm.at[pl.ds(tile_base(j0 + 2), T)],
                             rowa, sema)

        pltpu.make_async_copy(out_hbm.at[pl.ds(0, T)], rowb, semb).wait()
        carry = process(rowb, j0 + 1, carry)

        @pl.when(j0 + 3 < N_TILES)
        def _next_b():
            pltpu.async_copy(out_hbm.at[pl.ds(tile_base(j0 + 3), T)],
                             rowb, semb)

        return carry

    zeros0 = tuple(zero16 for _ in range(D // 16))
    carry = (jnp.int32(-1), zeros0, zeros0)
    carry = lax.fori_loop(0, N_PAIRS, pair_body, carry)
    flush(carry)

    pltpu.sync_copy(acc.at[pl.ds(0, SEG * D)], parts_hbm.at[w])


_sc_pool = pl.kernel(
    _sc_body,
    out_type=jax.ShapeDtypeStruct((NW, SEG * D), jnp.float32),
    mesh=plsc.VectorSubcoreMesh(core_axis_name="c", subcore_axis_name="s"),
    compiler_params=pltpu.CompilerParams(needs_layout_passes=False),
    scratch_types=[
        pltpu.VMEM((T, D), jnp.float32),
        pltpu.VMEM((T, D), jnp.float32),
        pltpu.VMEM((CHUNK,), jnp.int32),
        pltpu.VMEM((CHUNK,), jnp.int32),
        pltpu.VMEM((ACC_ROWS * D,), jnp.float32),
        pltpu.SemaphoreType.DMA,
        pltpu.SemaphoreType.DMA,
    ],
)


def _tc_body(zv_ref, bat_ref, rows_ref, o_ref):
    i = pl.program_id(0)
    zvb = zv_ref[0, 0, :]
    btb = bat_ref[0, 0, :]
    sv = btb * 2 + jnp.where(zvb == 100, 1, 0)
    w0 = jnp.minimum((btb[0] * 2) // 8 * 8, SEG - W)
    w0 = pl.multiple_of(w0, 8)
    svw = sv - w0
    rows_bf = rows_ref[...].astype(jnp.bfloat16)

    @pl.when(i == 0)
    def _init():
        o_ref[...] = jnp.zeros((SEG, D), jnp.float32)

    oh = (lax.broadcasted_iota(jnp.int32, (W, RB), 0)
          == svw[None, :]).astype(jnp.bfloat16)
    o_ref[pl.ds(w0, W), :] += jnp.dot(oh, rows_bf,
                                      preferred_element_type=jnp.float32)

    @pl.when(jnp.any(svw >= W))
    def _fallback():
        ohf = ((lax.broadcasted_iota(jnp.int32, (SEG, RB), 0) == sv[None, :])
               & (svw >= W)[None, :]).astype(jnp.bfloat16)
        o_ref[...] += jnp.dot(ohf, rows_bf,
                              preferred_element_type=jnp.float32)


_tc_pool = pl.pallas_call(
    _tc_body,
    grid=(NB,),
    in_specs=[
        pl.BlockSpec((1, 1, RB), lambda i: (i, 0, 0)),
        pl.BlockSpec((1, 1, RB), lambda i: (i, 0, 0)),
        pl.BlockSpec((RB, D), lambda i: (i, 0)),
    ],
    out_specs=pl.BlockSpec((SEG, D), lambda i: (0, 0)),
    out_shape=jax.ShapeDtypeStruct((SEG, D), jnp.float32),
)


def _combine_body(p_ref, t_ref, o_ref):
    acc = t_ref[...]
    for i in range(NW):
        acc = acc + p_ref[i]
    o_ref[...] = acc


_combine = pl.pallas_call(
    _combine_body,
    out_shape=jax.ShapeDtypeStruct((SEG * D,), jnp.float32),
)


def kernel(out, zv, x_rv_batch):
    zv32 = zv.astype(jnp.int32)
    bat32 = x_rv_batch.astype(jnp.int32)
    tc_part = _tc_pool(zv32[:NTC].reshape(NB, 1, RB),
                       bat32[:NTC].reshape(NB, 1, RB),
                       out[:NTC])
    parts = _sc_pool(out, zv32, bat32)
    summed = _combine(parts, tc_part.reshape(SEG * D))
    return summed.reshape(B, 2 * D)
